# baseline (speedup 1.0000x reference)
import jax
import jax.numpy as jnp
from jax import lax
from jax.experimental import pallas as pl
from jax.experimental.pallas import tpu as pltpu
from jax.experimental.pallas import tpu_sc as plsc

NUM_LABELS = 1000000
EMBED = 32
SLOTS = 5
BATCH = 16384
BFLAT = BATCH * SLOTS

_info = plsc.get_sparse_core_info()
NC, NS, LANES = _info.num_cores, _info.num_subcores, _info.num_lanes
NW = NC * NS
B_PER_W = BFLAT // NW
CHUNK = 128
NCHUNK = B_PER_W // CHUNK
GROUPS = B_PER_W // SLOTS


def _body(idx_hbm, table_hbm, wfull_hbm, out_hbm, idx_v, rows_v, w_v, sem):
    wid = lax.axis_index("s") * NC + lax.axis_index("c")
    base = wid * B_PER_W

    pltpu.sync_copy(idx_hbm.at[wid], idx_v)
    pltpu.sync_copy(wfull_hbm, w_v)

    copies = []
    for c in range(NCHUNK):
        copies.append(
            pltpu.async_copy(
                table_hbm.at[idx_v.at[c]],
                rows_v.at[pl.ds(c * CHUNK, CHUNK)],
                sem,
            )
        )
    for cp in copies:
        cp.wait()

    wvec = [w_v[pl.ds(16 * k, 16)] for k in range(2 * SLOTS)]

    @plsc.parallel_loop(0, GROUPS, step=1)
    def _(g):
        r0 = g * SLOTS
        for j in range(SLOTS):
            for h in range(2):
                rows_v[r0 + j, pl.ds(16 * h, 16)] = (
                    rows_v[r0 + j, pl.ds(16 * h, 16)] * wvec[2 * j + h]
                )

    pltpu.sync_copy(rows_v, out_hbm.at[pl.ds(base, B_PER_W)])


@jax.jit
def _gather_scale(idx2d, table, wfull):
    mesh = plsc.VectorSubcoreMesh(core_axis_name="c", subcore_axis_name="s")
    return pl.kernel(
        _body,
        out_type=jax.ShapeDtypeStruct((BFLAT, EMBED), jnp.float32),
        mesh=mesh,
        scratch_types=[
            pltpu.VMEM((NCHUNK, CHUNK), jnp.int32),
            pltpu.VMEM((B_PER_W, EMBED), jnp.float32),
            pltpu.VMEM((2 * SLOTS * 16,), jnp.float32),
            pltpu.SemaphoreType.DMA,
        ],
        compiler_params=pltpu.CompilerParams(use_tc_tiling_on_sc=False),
    )(idx2d, table, wfull)


def kernel(x, table, weight):
    idx2d = x.astype(jnp.int32).reshape(NW, NCHUNK, CHUNK)
    wfull = jnp.repeat(weight.astype(jnp.float32), EMBED)
    out = _gather_scale(idx2d, table, wfull)
    return out.reshape(BATCH, SLOTS * EMBED)
